# baseline (speedup 1.0000x reference)
import jax
import jax.numpy as jnp
from jax.experimental import pallas as pl
from jax.experimental.pallas import tpu as pltpu

_N = 8192
_HALF = 4096
_D = 128
_MB = 128
_NB = _N // _MB


def _gcn_kernel(e_ref, x_ref, wr_ref, wd_ref, b_ref, o_ref, y_scr, dis_scr):
    i = pl.program_id(0)

    @pl.when(i < _NB)
    def _phase1():
        s = jnp.sum(e_ref[...], axis=1, keepdims=True) + 1.0
        dis = jnp.where(s > 0.0, jax.lax.rsqrt(s), 0.0)
        dis_scr[pl.ds(i * _MB, _MB), :] = dis
        w = jnp.where(i * _MB < _HALF, wr_ref[...], wd_ref[...])
        y_scr[pl.ds(i * _MB, _MB), :] = dis * jnp.dot(
            x_ref[...], w, preferred_element_type=jnp.float32
        )

    @pl.when(i >= _NB)
    def _phase2():
        j = i - _NB
        z = jnp.dot(e_ref[...], y_scr[...], preferred_element_type=jnp.float32)
        o = (
            dis_scr[pl.ds(j * _MB, _MB), :] * (z + y_scr[pl.ds(j * _MB, _MB), :])
            + b_ref[...]
        )
        o_ref[...] = jnp.where(o >= 0.0, o, 0.01 * o)


def kernel(x, edge_index, weightr, weightd, bias):
    out = pl.pallas_call(
        _gcn_kernel,
        grid=(2 * _NB,),
        in_specs=[
            pl.BlockSpec((_MB, _N), lambda i: (i % _NB, 0)),
            pl.BlockSpec((_MB, _D), lambda i: (jnp.where(i < _NB, i, 0), 0)),
            pl.BlockSpec((_D, _D), lambda i: (0, 0)),
            pl.BlockSpec((_D, _D), lambda i: (0, 0)),
            pl.BlockSpec((1, _D), lambda i: (0, 0)),
        ],
        out_specs=pl.BlockSpec(
            (_MB, _D), lambda i: (jnp.where(i < _NB, 0, i - _NB), 0)
        ),
        out_shape=jax.ShapeDtypeStruct((_N, _D), jnp.float32),
        scratch_shapes=[
            pltpu.VMEM((_N, _D), jnp.float32),
            pltpu.VMEM((_N, 1), jnp.float32),
        ],
    )(edge_index, x, weightr, weightd, bias.reshape(1, _D))
    return out


# speedup vs baseline: 1.0160x; 1.0160x over previous
import functools

import jax
import jax.numpy as jnp
from jax import lax
from jax.experimental import pallas as pl
from jax.experimental.pallas import tpu as pltpu
from jax.experimental.pallas import tpu_sc as plsc

_N = 8192
_HALF = 4096
_D = 128
_MB = 256
_SC_ROWS = 2048
_R_TC = _N - _SC_ROWS
_NB1 = _R_TC // _MB
_NB2 = _N // _MB

_NC = 2
_NS = 16
_NW = _NC * _NS
_RPW = _SC_ROWS // _NW
_CH = 4
_NCH = _RPW // _CH


def _tc_deg_kernel(e_ref, dis_ref):
    s = jnp.sum(e_ref[...], axis=1, keepdims=True) + 1.0
    dis_ref[...] = jnp.where(s > 0.0, jax.lax.rsqrt(s), 0.0)


def _row_sum16(buf_ref, b, r):
    def body(j, accs):
        a0, a1, a2, a3 = accs
        base = j * 64
        a0 = a0 + buf_ref[b, r, pl.ds(base, 16)]
        a1 = a1 + buf_ref[b, r, pl.ds(base + 16, 16)]
        a2 = a2 + buf_ref[b, r, pl.ds(base + 32, 16)]
        a3 = a3 + buf_ref[b, r, pl.ds(base + 48, 16)]
        return a0, a1, a2, a3
    z = jnp.zeros((16,), jnp.float32)
    a0, a1, a2, a3 = lax.fori_loop(0, _N // 64, body, (z, z, z, z))
    return (a0 + a1) + (a2 + a3)


def _sc_deg_kernel(e_hbm, out_hbm, buf_ref, acc_ref, sem0, sem1):
    wid = lax.axis_index("s") * _NC + lax.axis_index("c")
    row0 = _R_TC + wid * _RPW
    sems = (sem0, sem1)

    def start(ch, b):
        return pltpu.async_copy(
            e_hbm.at[pl.ds(row0 + ch * _CH, _CH)], buf_ref.at[b], sems[b]
        )

    handles = {0: start(0, 0), 1: start(1, 1)}
    for ch in range(_NCH):
        b = ch % 2
        handles.pop(ch).wait()
        for r in range(_CH):
            acc_ref[ch * _CH + r, :] = _row_sum16(buf_ref, b, r)
        nxt = ch + 2
        if nxt < _NCH:
            handles[nxt] = start(nxt, b)
    pltpu.sync_copy(acc_ref, out_hbm.at[pl.ds(wid * _RPW, _RPW)])


def _tc_main_kernel(e_ref, x_ref, wr_ref, wd_ref, b_ref, dtc_ref, scp_ref,
                    o_ref, y_scr, dis_scr):
    p = pl.program_id(0)

    @pl.when(p == 0)
    def _project():
        s_sc = jnp.sum(scp_ref[...], axis=1, keepdims=True) + 1.0
        dis_sc = jnp.where(s_sc > 0.0, jax.lax.rsqrt(s_sc), 0.0)
        dis_scr[0:_R_TC, :] = dtc_ref[...]
        dis_scr[_R_TC:_N, :] = dis_sc
        top = jnp.dot(x_ref[0:_HALF, :], wr_ref[...],
                      preferred_element_type=jnp.float32)
        y_scr[0:_HALF, :] = dtc_ref[0:_HALF, :] * top
        mid = jnp.dot(x_ref[_HALF:_R_TC, :], wd_ref[...],
                      preferred_element_type=jnp.float32)
        y_scr[_HALF:_R_TC, :] = dtc_ref[_HALF:_R_TC, :] * mid
        tail = jnp.dot(x_ref[_R_TC:_N, :], wd_ref[...],
                       preferred_element_type=jnp.float32)
        y_scr[_R_TC:_N, :] = dis_sc * tail

    @pl.when(p > 0)
    def _matmul():
        j = p - 1
        z = jnp.dot(e_ref[...], y_scr[...], preferred_element_type=jnp.float32)
        o = (
            dis_scr[pl.ds(j * _MB, _MB), :] * (z + y_scr[pl.ds(j * _MB, _MB), :])
            + b_ref[...]
        )
        o_ref[...] = jnp.where(o >= 0.0, o, 0.01 * o)


_sc_deg = functools.partial(
    pl.kernel,
    mesh=plsc.VectorSubcoreMesh(core_axis_name="c", subcore_axis_name="s"),
    out_type=jax.ShapeDtypeStruct((_SC_ROWS, 16), jnp.float32),
    scratch_types=[
        pltpu.VMEM((2, _CH, _N), jnp.float32),
        pltpu.VMEM((_RPW, 16), jnp.float32),
        pltpu.SemaphoreType.DMA,
        pltpu.SemaphoreType.DMA,
    ],
)(_sc_deg_kernel)


def kernel(x, edge_index, weightr, weightd, bias):
    scpart = _sc_deg(edge_index)

    dis_tc = pl.pallas_call(
        _tc_deg_kernel,
        grid=(_NB1,),
        in_specs=[pl.BlockSpec((_MB, _N), lambda i: (i, 0))],
        out_specs=pl.BlockSpec((_MB, 1), lambda i: (i, 0)),
        out_shape=jax.ShapeDtypeStruct((_R_TC, 1), jnp.float32),
    )(edge_index)

    out = pl.pallas_call(
        _tc_main_kernel,
        grid=(1 + _NB2,),
        in_specs=[
            pl.BlockSpec((_MB, _N), lambda p: (jnp.where(p == 0, 0, p - 1), 0)),
            pl.BlockSpec((_N, _D), lambda p: (0, 0)),
            pl.BlockSpec((_D, _D), lambda p: (0, 0)),
            pl.BlockSpec((_D, _D), lambda p: (0, 0)),
            pl.BlockSpec((1, _D), lambda p: (0, 0)),
            pl.BlockSpec((_R_TC, 1), lambda p: (0, 0)),
            pl.BlockSpec((_SC_ROWS, 16), lambda p: (0, 0)),
        ],
        out_specs=pl.BlockSpec(
            (_MB, _D), lambda p: (jnp.where(p == 0, 0, p - 1), 0)
        ),
        out_shape=jax.ShapeDtypeStruct((_N, _D), jnp.float32),
        scratch_shapes=[
            pltpu.VMEM((_N, _D), jnp.float32),
            pltpu.VMEM((_N, 1), jnp.float32),
        ],
    )(edge_index, x, weightr, weightd, bias.reshape(1, _D), dis_tc, scpart)
    return out
